# Initial kernel scaffold; baseline (speedup 1.0000x reference)
import jax
import jax.numpy as jnp
from jax.experimental import pallas as pl


def kernel(x1, edge_index1, batch1, x2, edge_index2, batch2, cell, params):
    raise NotImplementedError("write your pallas kernel here")


# speedup vs baseline: 8.6794x; 8.6794x over previous
import functools

import jax
import jax.numpy as jnp
import numpy as np
from jax import lax
from jax.experimental import pallas as pl
from jax.experimental.pallas import tpu as pltpu
from jax.experimental.pallas import tpu_sc as plsc

N = 50000
E = 800000
B = 512
DIM = 32
DXD = 78
NC = 2
NS = 16
K = 128

_EBLK = -(-E // (NS * K))
_EBLK += _EBLK % 2
_CH = 28
_EPAD = NS * _EBLK * K
_PBLK = -(-N // (NS * K))
_PBLK += _PBLK % 2
_PPAD = NS * _PBLK * K
_STR = 3128

_R = 2 * N // 4
_BR = 5000
_BN = 5000

_HIGH = lax.Precision.HIGHEST

_mesh = plsc.VectorSubcoreMesh(core_axis_name="c", subcore_axis_name="s")
_sc_params = pltpu.CompilerParams(use_tc_tiling_on_sc=False)


@functools.partial(
    pl.kernel,
    out_type=jax.ShapeDtypeStruct((NC, N, DIM), jnp.float32),
    mesh=_mesh,
    compiler_params=_sc_params,
    scratch_types=[
        pltpu.VMEM_SHARED((N + 8, DIM), jnp.float32),
        pltpu.VMEM((_CH, K), jnp.int32),
        pltpu.VMEM((_CH, K), jnp.int32),
        pltpu.VMEM((K, DIM), jnp.float32),
        pltpu.SemaphoreType.DMA,
    ],
)
def _sc_aggregate(y_hbm, src_hbm, dst_hbm, out_hbm, acc, srcv, dstv, rows, sem):
    c = lax.axis_index("c")
    s = lax.axis_index("s")
    off = pl.multiple_of(jnp.minimum(s * _STR, N - _STR), 8)
    pltpu.sync_copy(
        y_hbm.at[pl.ds(c * N + off, _STR)], acc.at[pl.ds(off, _STR)]
    )
    plsc.subcore_barrier()

    @pl.loop(0, _EBLK // _CH)
    def _(j):
        pltpu.sync_copy(src_hbm.at[c, s, pl.ds(j * _CH, _CH)], srcv)
        pltpu.sync_copy(dst_hbm.at[c, s, pl.ds(j * _CH, _CH)], dstv)

        @pl.loop(0, _CH)
        def _(i):
            pltpu.async_copy(y_hbm.at[srcv.at[i]], rows, sem).wait()
            pltpu.sync_copy(rows, acc.at[dstv.at[i]], add=True)

    plsc.subcore_barrier()
    pltpu.sync_copy(acc.at[pl.ds(off, _STR)], out_hbm.at[c, pl.ds(off, _STR)])


@functools.partial(
    pl.kernel,
    out_type=jax.ShapeDtypeStruct((NC, B, DIM), jnp.float32),
    mesh=_mesh,
    compiler_params=_sc_params,
    scratch_types=[
        pltpu.VMEM_SHARED((B + 8, DIM), jnp.float32),
        pltpu.VMEM((_PBLK, K), jnp.int32),
        pltpu.VMEM((_PBLK, K), jnp.int32),
        pltpu.VMEM((K, DIM), jnp.float32),
        pltpu.SemaphoreType.DMA,
    ],
)
def _sc_pool(h_hbm, src_hbm, dst_hbm, zero_hbm, out_hbm, acc, srcv, dstv, rows, sem):
    c = lax.axis_index("c")
    s = lax.axis_index("s")
    pltpu.sync_copy(src_hbm.at[c, s], srcv)
    pltpu.sync_copy(dst_hbm.at[c, s], dstv)

    @pl.when(s == 0)
    def _():
        pltpu.sync_copy(zero_hbm, acc)

    plsc.subcore_barrier()

    @pl.loop(0, _PBLK)
    def _(i):
        pltpu.async_copy(h_hbm.at[srcv.at[i]], rows, sem).wait()
        pltpu.sync_copy(rows, acc.at[dstv.at[i]], add=True)

    plsc.subcore_barrier()

    @pl.when(s == 0)
    def _():
        pltpu.sync_copy(acc.at[pl.ds(0, B)], out_hbm.at[c])


def _tc_first_body(x1_ref, x2_ref, w_ref, o_ref):
    w = w_ref[...]
    o_ref[0] = jnp.dot(x1_ref[...], w, precision=_HIGH,
                       preferred_element_type=jnp.float32)
    o_ref[1] = jnp.dot(x2_ref[...], w, precision=_HIGH,
                       preferred_element_type=jnp.float32)


def _tc_first(x1, x2, w1):
    return pl.pallas_call(
        _tc_first_body,
        grid=(N // _BN,),
        in_specs=[
            pl.BlockSpec((_BN, DXD), lambda i: (i, 0)),
            pl.BlockSpec((_BN, DXD), lambda i: (i, 0)),
            pl.BlockSpec((DXD, DIM), lambda i: (0, 0)),
        ],
        out_specs=pl.BlockSpec((2, _BN, DIM), lambda i: (0, i, 0)),
        out_shape=jax.ShapeDtypeStruct((2, N, DIM), jnp.float32),
    )(x1, x2, w1)


def _tc_layer_body(last, ys_ref, w2_ref, w1n_ref, b1_ref, b2_ref, g_ref,
                   bt_ref, o_ref):
    u = jnp.maximum(ys_ref[...] + b1_ref[...], 0.0)
    t = jnp.dot(u, w2_ref[...], precision=_HIGH,
                preferred_element_type=jnp.float32) + b2_ref[...]
    t = jnp.maximum(t, 0.0)
    h = t * g_ref[...] + bt_ref[...]
    if last:
        o_ref[...] = h
    else:
        o_ref[...] = jnp.dot(h, w1n_ref[...], precision=_HIGH,
                             preferred_element_type=jnp.float32)


def _tc_layer(ysp, w2p, w1np, b1p, b2p, gp, btp, last):
    return pl.pallas_call(
        functools.partial(_tc_layer_body, last),
        grid=(_R // _BR,),
        in_specs=[
            pl.BlockSpec((_BR, 128), lambda i: (i, 0)),
            pl.BlockSpec((128, 128), lambda i: (0, 0)),
            pl.BlockSpec((128, 128), lambda i: (0, 0)),
            pl.BlockSpec((1, 128), lambda i: (0, 0)),
            pl.BlockSpec((1, 128), lambda i: (0, 0)),
            pl.BlockSpec((1, 128), lambda i: (0, 0)),
            pl.BlockSpec((1, 128), lambda i: (0, 0)),
        ],
        out_specs=pl.BlockSpec((_BR, 128), lambda i: (i, 0)),
        out_shape=jax.ShapeDtypeStruct((_R, 128), jnp.float32),
    )(ysp, w2p, w1np, b1p, b2p, gp, btp)


def _tc_head_body(g_ref, cell_ref, wf_ref, bf_ref, wr1_ref, br1_ref, wr2_ref,
                  br2_ref, wr3_ref, br3_ref, wf1_ref, bf1_ref, wf2_ref,
                  bf2_ref, wo_ref, bo_ref, o_ref):
    def mm(a, b):
        return jnp.dot(a, b, precision=_HIGH, preferred_element_type=jnp.float32)

    v = jnp.maximum(mm(g_ref[...], wf_ref[...]) + bf_ref[...], 0.0)
    cellp = cell_ref[...]
    nrm = jnp.sqrt(jnp.sum(cellp * cellp, axis=1, keepdims=True))
    cn = cellp / jnp.maximum(nrm, 1e-12)
    c1 = jnp.maximum(mm(cn, wr1_ref[...]) + br1_ref[...], 0.0)
    c2 = jnp.maximum(mm(c1, wr2_ref[...]) + br2_ref[...], 0.0)
    c3 = mm(c2, wr3_ref[...]) + br3_ref[...]
    xc = jnp.concatenate([v[:B], v[B:], c3], axis=1)
    f1 = jnp.maximum(mm(xc, wf1_ref[...]) + bf1_ref[...], 0.0)
    f2 = jnp.maximum(mm(f1, wf2_ref[...]) + bf2_ref[...], 0.0)
    o_ref[...] = mm(f2, wo_ref[...]) + bo_ref[...]


def _tc_head(g2, cellp, args):
    return pl.pallas_call(
        _tc_head_body,
        out_shape=jax.ShapeDtypeStruct((B, 128), jnp.float32),
    )(g2, cellp, *args)


def _prep_edges(ei, c):
    pad = _EPAD - E
    src = jnp.concatenate([ei[0] + c * N, jnp.full((pad,), c * N, jnp.int32)])
    dst = jnp.concatenate([ei[1], jnp.full((pad,), N, jnp.int32)])
    return src.reshape(NS, _EBLK, K), dst.reshape(NS, _EBLK, K)


def _prep_pool(batch, c):
    pad = _PPAD - N
    src = jnp.concatenate(
        [jnp.arange(N, dtype=jnp.int32) + c * N, jnp.full((pad,), c * N, jnp.int32)]
    )
    dst = jnp.concatenate([batch, jnp.full((pad,), B, jnp.int32)])
    return src.reshape(NS, _PBLK, K), dst.reshape(NS, _PBLK, K)


def _kron4(w):
    return jnp.kron(jnp.eye(4, dtype=jnp.float32), w)


def _tile4(v):
    return jnp.tile(v, 4).reshape(1, 128)


def kernel(x1, edge_index1, batch1, x2, edge_index2, batch2, cell, params):
    gin = params["gin"]
    inv = np.float32(1.0 / np.sqrt(1.0 + 1e-5))

    s1, d1 = _prep_edges(edge_index1, 0)
    s2, d2 = _prep_edges(edge_index2, 1)
    src_all = jnp.stack([s1, s2])
    dst_all = jnp.stack([d1, d2])
    ps1, pd1 = _prep_pool(batch1, 0)
    ps2, pd2 = _prep_pool(batch2, 1)
    psrc = jnp.stack([ps1, ps2])
    pdst = jnp.stack([pd1, pd2])
    zeros = jnp.zeros((B + 8, DIM), jnp.float32)

    packed = []
    for l in range(5):
        lp = gin[l]
        w1n = _kron4(gin[l + 1]["W1"]) if l < 4 else jnp.zeros((128, 128), jnp.float32)
        packed.append((
            _kron4(lp["W2"]), w1n, _tile4(lp["b1"]), _tile4(lp["b2"]),
            _tile4(lp["gamma"] * inv), _tile4(lp["beta"]),
        ))

    y = _tc_first(x1, x2, gin[0]["W1"])
    for l in range(5):
        ys = _sc_aggregate(y.reshape(2 * N, DIM), src_all, dst_all)
        w2p, w1np, b1p, b2p, gp, btp = packed[l]
        out = _tc_layer(ys.reshape(_R, 128), w2p, w1np, b1p, b2p, gp, btp,
                        last=(l == 4))
        y = out.reshape(2, N, DIM)

    g = _sc_pool(y.reshape(2 * N, DIM), psrc, pdst, zeros)

    wf, bf = params["fc_xd"]
    wr1, br1 = params["red1"]
    wr2, br2 = params["red2"]
    wr3, br3 = params["red3"]
    wf1, bf1 = params["fc1"]
    wf2, bf2 = params["fc2"]
    wo, bo = params["out"]
    cellp = jnp.pad(cell, ((0, 0), (0, 1024 - cell.shape[1])))
    wr1p = jnp.pad(wr1, ((0, 1024 - wr1.shape[0]), (0, 0)))
    wop = jnp.pad(wo, ((0, 0), (0, 126)))
    bop = jnp.pad(bo, ((0, 126),)).reshape(1, 128)
    args = (wf, bf.reshape(1, -1), wr1p, br1.reshape(1, -1),
            wr2, br2.reshape(1, -1), wr3, br3.reshape(1, -1),
            wf1, bf1.reshape(1, -1), wf2, bf2.reshape(1, -1), wop, bop)
    out = _tc_head(g.reshape(2 * B, DIM), cellp, args)
    return out[:, :2]
